# baseline (speedup 1.0000x reference)
import functools

import jax
import jax.numpy as jnp
from jax import lax
from jax.experimental import pallas as pl
from jax.experimental.pallas import tpu as pltpu
from jax.experimental.pallas import tpu_sc as plsc

NUM_ENTITIES = 100000
NUM_RELATIONS = 1000
EMBED_DIM = 64
BATCH = 1024

_NC = 2
_NS = 16
_NW = _NC * _NS
_LANES = 16
_BPW = BATCH // _NW

_TB = 32
_NJ = BATCH // _TB
_VC = 2048
_NFULL = NUM_ENTITIES // _VC
_VEDGE = NUM_ENTITIES - _NFULL * _VC


def _gather_mul_body(s_hbm, p_hbm, o_hbm, ent_hbm, rel_hbm, out_hbm,
                     s_v, p_v, o_v, es_v, eo_v, rp_v, sem):
    wid = lax.axis_index("s") * _NC + lax.axis_index("c")
    base = wid * _BPW
    pltpu.sync_copy(s_hbm.at[pl.ds(base, _BPW)], s_v)
    pltpu.sync_copy(p_hbm.at[pl.ds(base, _BPW)], p_v)
    pltpu.sync_copy(o_hbm.at[pl.ds(base, _BPW)], o_v)
    c_s = pltpu.async_copy(ent_hbm.at[s_v], es_v, sem)
    c_o = pltpu.async_copy(ent_hbm.at[o_v], eo_v, sem)
    c_p = pltpu.async_copy(rel_hbm.at[p_v], rp_v, sem)
    c_s.wait()
    c_o.wait()
    c_p.wait()
    for i in range(_BPW):
        for j in range(EMBED_DIM // _LANES):
            sl = (i, pl.ds(j * _LANES, _LANES))
            r = rp_v[sl]
            es_v[sl] = es_v[sl] * r
            eo_v[sl] = eo_v[sl] * r
    pltpu.sync_copy(es_v, out_hbm.at[pl.ds(base, _BPW)])
    pltpu.sync_copy(eo_v, out_hbm.at[pl.ds(BATCH + base, _BPW)])


_gather_mul = functools.partial(
    pl.kernel,
    mesh=plsc.VectorSubcoreMesh(core_axis_name="c", subcore_axis_name="s"),
    compiler_params=pltpu.CompilerParams(use_tc_tiling_on_sc=False),
    out_type=jax.ShapeDtypeStruct((2 * BATCH, EMBED_DIM), jnp.float32),
    scratch_types=[
        pltpu.VMEM((_BPW,), jnp.int32),
        pltpu.VMEM((_BPW,), jnp.int32),
        pltpu.VMEM((_BPW,), jnp.int32),
        pltpu.VMEM((_BPW, EMBED_DIM), jnp.float32),
        pltpu.VMEM((_BPW, EMBED_DIM), jnp.float32),
        pltpu.VMEM((_BPW, EMBED_DIM), jnp.float32),
        pltpu.SemaphoreType.DMA,
    ],
)(_gather_mul_body)


def _score_body(q_ref, e_hbm, out_ref, evmem, esem):
    p = pl.program_id(0)
    j = pl.program_id(1)

    @pl.when(jnp.logical_and(p == 0, j == 0))
    def _load_e():
        pltpu.make_async_copy(e_hbm, evmem, esem).start()
        pltpu.make_async_copy(e_hbm, evmem, esem).wait()

    qv = q_ref[...]
    for c in range(_NFULL + 1):
        off = c * _VC
        w = _VC if c < _NFULL else _VEDGE
        out_ref[0, :, pl.ds(off, w)] = lax.dot_general(
            qv, evmem[:, pl.ds(off, w)],
            dimension_numbers=(((1,), (0,)), ((), ())),
            preferred_element_type=jnp.float32,
            precision=lax.Precision.DEFAULT,
        )


def kernel(s, p, o, entity_embedding, relation_embedding):
    s = s.astype(jnp.int32)
    p = p.astype(jnp.int32)
    o = o.astype(jnp.int32)
    q = _gather_mul(s, p, o, entity_embedding, relation_embedding)
    scores = pl.pallas_call(
        _score_body,
        grid=(2, _NJ),
        in_specs=[
            pl.BlockSpec((_TB, EMBED_DIM), lambda p, j: (p * _NJ + j, 0)),
            pl.BlockSpec(memory_space=pl.ANY),
        ],
        out_specs=pl.BlockSpec((1, _TB, NUM_ENTITIES), lambda p, j: (p, j, 0)),
        out_shape=jax.ShapeDtypeStruct((2, BATCH, NUM_ENTITIES), jnp.float32),
        scratch_shapes=[
            pltpu.VMEM((EMBED_DIM, NUM_ENTITIES), jnp.float32),
            pltpu.SemaphoreType.DMA,
        ],
        compiler_params=pltpu.CompilerParams(
            vmem_limit_bytes=60000 * 1024,
        ),
    )(q, entity_embedding.T)
    return scores


# speedup vs baseline: 1.1308x; 1.1308x over previous
import functools

import jax
import jax.numpy as jnp
from jax import lax
from jax.experimental import pallas as pl
from jax.experimental.pallas import tpu as pltpu
from jax.experimental.pallas import tpu_sc as plsc

NUM_ENTITIES = 100000
NUM_RELATIONS = 1000
EMBED_DIM = 64
BATCH = 1024

_NC = 2
_NS = 16
_NW = _NC * _NS
_LANES = 16
_BPW = BATCH // _NW

_VC = 2048
_NV = -(-NUM_ENTITIES // _VC)


def _gather_mul_body(s_hbm, p_hbm, o_hbm, ent_hbm, rel_hbm, out_hbm,
                     s_v, p_v, o_v, es_v, eo_v, rp_v, sem):
    wid = lax.axis_index("s") * _NC + lax.axis_index("c")
    base = wid * _BPW
    pltpu.sync_copy(s_hbm.at[pl.ds(base, _BPW)], s_v)
    pltpu.sync_copy(p_hbm.at[pl.ds(base, _BPW)], p_v)
    pltpu.sync_copy(o_hbm.at[pl.ds(base, _BPW)], o_v)
    c_s = pltpu.async_copy(ent_hbm.at[s_v], es_v, sem)
    c_o = pltpu.async_copy(ent_hbm.at[o_v], eo_v, sem)
    c_p = pltpu.async_copy(rel_hbm.at[p_v], rp_v, sem)
    c_s.wait()
    c_o.wait()
    c_p.wait()
    for i in range(_BPW):
        for j in range(EMBED_DIM // _LANES):
            sl = (i, pl.ds(j * _LANES, _LANES))
            r = rp_v[sl]
            es_v[sl] = es_v[sl] * r
            eo_v[sl] = eo_v[sl] * r
    pltpu.sync_copy(es_v, out_hbm.at[pl.ds(base, _BPW)])
    pltpu.sync_copy(eo_v, out_hbm.at[pl.ds(BATCH + base, _BPW)])


_gather_mul = functools.partial(
    pl.kernel,
    mesh=plsc.VectorSubcoreMesh(core_axis_name="c", subcore_axis_name="s"),
    compiler_params=pltpu.CompilerParams(use_tc_tiling_on_sc=False),
    out_type=jax.ShapeDtypeStruct((2 * BATCH, EMBED_DIM), jnp.float32),
    scratch_types=[
        pltpu.VMEM((_BPW,), jnp.int32),
        pltpu.VMEM((_BPW,), jnp.int32),
        pltpu.VMEM((_BPW,), jnp.int32),
        pltpu.VMEM((_BPW, EMBED_DIM), jnp.float32),
        pltpu.VMEM((_BPW, EMBED_DIM), jnp.float32),
        pltpu.VMEM((_BPW, EMBED_DIM), jnp.float32),
        pltpu.SemaphoreType.DMA,
    ],
)(_gather_mul_body)


def _score_body(q_ref, e_ref, out_ref):
    out_ref[...] = lax.dot_general(
        q_ref[...], e_ref[...],
        dimension_numbers=(((1,), (1,)), ((), ())),
        preferred_element_type=jnp.float32,
        precision=lax.Precision.DEFAULT,
    )


def kernel(s, p, o, entity_embedding, relation_embedding):
    s = s.astype(jnp.int32)
    p = p.astype(jnp.int32)
    o = o.astype(jnp.int32)
    q = _gather_mul(s, p, o, entity_embedding, relation_embedding)
    scores = pl.pallas_call(
        _score_body,
        grid=(_NV,),
        in_specs=[
            pl.BlockSpec((2 * BATCH, EMBED_DIM), lambda v: (0, 0)),
            pl.BlockSpec((_VC, EMBED_DIM), lambda v: (v, 0)),
        ],
        out_specs=pl.BlockSpec((2 * BATCH, _VC), lambda v: (0, v)),
        out_shape=jax.ShapeDtypeStruct((2 * BATCH, NUM_ENTITIES), jnp.float32),
        compiler_params=pltpu.CompilerParams(
            vmem_limit_bytes=100 * 1024 * 1024,
        ),
    )(q, entity_embedding)
    return scores.reshape(2, BATCH, NUM_ENTITIES)
